# baseline (speedup 1.0000x reference)
import jax
import jax.numpy as jnp
import numpy as np
from jax import lax
from jax.experimental import pallas as pl
from jax.experimental.pallas import tpu as pltpu
from jax.experimental.pallas import tpu_sc as plsc

NV_PREV = 40962
NV = 163842
NV_PAD = NV - NV_PREV
NF_PREV = 81920

NC, NS, L = 2, 16, 16
NW = NC * NS

BIAS = 128
ROWBUF = 41216
EXT_LEN = 126976
FILL_PER_TILE = EXT_LEN // NS
OUT_PER_TILE = EXT_LEN // NW
FACES_PER_TILE = NF_PREV // NS
SCAT_ROWS = FACES_PER_TILE // 128

K = 6144
NCHUNK = NV_PAD // K
ROW_PAIRS_PER_W = (256 // 2) // NW
VOLD = 40960

_SELF0 = int(np.int32(np.uint32(((40960 + BIAS) << 16) | (40960 + BIAS))))
_SELF1 = int(np.int32(np.uint32(((40961 + BIAS) << 16) | (40961 + BIAS))))

_MESH = plsc.VectorSubcoreMesh(
    core_axis_name="c", subcore_axis_name="s", num_cores=NC, num_subcores=NS)

_CPARAMS = pltpu.CompilerParams(
    needs_layout_passes=False, use_tc_tiling_on_sc=True)


def _pairs_body(f_ref, i0_ref, i1_ref, i2_ref, out_ref,
                fbuf, ibuf, vb01, vb12, dbuf, zbuf, shared, sem):
    cid = lax.axis_index("c")
    sid = lax.axis_index("s")
    iota = lax.iota(jnp.int32, L)
    zeros = jnp.zeros((L,), jnp.int32)
    fbase = sid * FACES_PER_TILE

    pltpu.sync_copy(f_ref.at[pl.ds(fbase * 3, FACES_PER_TILE * 3)], fbuf)

    def vloop(i, carry):
        flat = (i * L + iota) * 3
        f0 = plsc.load_gather(fbuf, [flat])
        f1 = plsc.load_gather(fbuf, [flat + 1])
        f2 = plsc.load_gather(fbuf, [flat + 2])
        v01 = (f0 + BIAS) | ((f1 + BIAS) << 16)
        v12 = (f1 + BIAS) | ((f2 + BIAS) << 16)
        r = i // 8
        c = (i % 8) * L
        vb01[r, pl.ds(c, L)] = v01
        vb12[r, pl.ds(c, L)] = v12
        return carry

    lax.fori_loop(0, FACES_PER_TILE // L, vloop, 0)

    def zloop(i, carry):
        zbuf[pl.ds(i * L, L)] = zeros
        return carry

    lax.fori_loop(0, FILL_PER_TILE // L, zloop, 0)

    @pl.when(sid == 0)
    def _():
        patch = jnp.where(iota == 0, jnp.int32(_SELF0),
                          jnp.where(iota == 1, jnp.int32(_SELF1), 0))
        zbuf[pl.ds(0, L)] = patch

    pltpu.sync_copy(zbuf, shared.at[pl.ds(sid * FILL_PER_TILE, FILL_PER_TILE)])
    plsc.subcore_barrier()

    for idx_ref, vb in ((i0_ref, vb01), (i1_ref, vb12), (i2_ref, vb12)):
        pltpu.sync_copy(idx_ref.at[pl.ds(fbase, FACES_PER_TILE)], ibuf)

        def dloop(i, carry):
            dv = ibuf[pl.ds(i * L, L)] - (NV_PREV - 2)
            r = i // 8
            c = (i % 8) * L
            dbuf[r, pl.ds(c, L)] = dv
            return carry

        lax.fori_loop(0, FACES_PER_TILE // L, dloop, 0)
        descs = [pltpu.async_copy(vb.at[j], shared.at[dbuf.at[j]], sem)
                 for j in range(SCAT_ROWS)]
        for d in descs:
            d.wait()
        plsc.subcore_barrier()

    wid = sid * NC + cid
    obase = wid * OUT_PER_TILE
    pltpu.sync_copy(shared.at[pl.ds(obase, OUT_PER_TILE)],
                    zbuf.at[pl.ds(0, OUT_PER_TILE)])
    pltpu.sync_copy(zbuf.at[pl.ds(0, OUT_PER_TILE)],
                    out_ref.at[pl.ds(obase, OUT_PER_TILE)])


_pairs_call = pl.kernel(
    _pairs_body,
    out_type=jax.ShapeDtypeStruct((EXT_LEN,), jnp.int32),
    mesh=_MESH,
    compiler_params=_CPARAMS,
    scratch_types=[
        pltpu.VMEM((FACES_PER_TILE * 3,), jnp.int32),
        pltpu.VMEM((FACES_PER_TILE,), jnp.int32),
        pltpu.VMEM((SCAT_ROWS, 128), jnp.int32),
        pltpu.VMEM((SCAT_ROWS, 128), jnp.int32),
        pltpu.VMEM((SCAT_ROWS, 128), jnp.int32),
        pltpu.VMEM((FILL_PER_TILE,), jnp.int32),
        pltpu.VMEM_SHARED((EXT_LEN,), jnp.int32),
        pltpu.SemaphoreType.DMA,
    ],
)


def _main_body(x_ref, ext_ref, xt_ref, out_ref,
               rp, ib0, ib1, oa, ob, xtb,
               si0, si1, sol, sold):
    cid = lax.axis_index("c")
    sid = lax.axis_index("s")
    iota = lax.iota(jnp.int32, L)
    wid = sid * NC + cid
    ones = jnp.full((L,), 1.0, jnp.float32)
    mask16 = jnp.int32(0xFFFF)
    zrow = jnp.zeros((L,), jnp.int32)
    orow = jnp.full((L,), 1, jnp.int32)
    ibufs = (ib0, ib1)
    obufs = (oa, ob)
    isems = (si0, si1)

    pltpu.sync_copy(xt_ref, xtb)

    def group(k, carry):
        r0 = 2 * (wid * ROW_PAIRS_PER_W + k)
        b0, c0 = r0 // 64, r0 % 64
        rp[0, pl.ds(0, L)] = ones
        rp[1, pl.ds(0, L)] = ones
        pltpu.sync_copy(x_ref.at[b0, pl.ds(c0, 2), pl.ds(0, VOLD)],
                        rp.at[:, pl.ds(BIAS, VOLD)])
        pos = iota + (BIAS + VOLD)
        msk2 = iota < 2
        v0 = plsc.load_gather(xtb, [jnp.minimum(2 * r0 + iota, 511)])
        v1 = plsc.load_gather(xtb, [jnp.minimum(2 * r0 + 2 + iota, 511)])
        plsc.store_scatter(rp, [zrow, pos], v0, mask=msk2)
        plsc.store_scatter(rp, [orow, pos], v1, mask=msk2)
        od = pltpu.async_copy(rp.at[:, pl.ds(BIAS, VOLD)],
                              out_ref.at[b0, pl.ds(c0, 2), pl.ds(0, VOLD)],
                              sold)

        idescs = {}
        odescs = {}
        idescs[0] = pltpu.async_copy(ext_ref.at[pl.ds(0, K)], ibufs[0], isems[0])
        for c in range(NCHUNK):
            cur = c % 2
            if c + 1 < NCHUNK:
                idescs[c + 1] = pltpu.async_copy(
                    ext_ref.at[pl.ds((c + 1) * K, K)],
                    ibufs[(c + 1) % 2], isems[(c + 1) % 2])
            idescs[c].wait()
            if c >= 2:
                odescs[c - 2].wait()
            ib = ibufs[cur]
            o2 = obufs[cur]

            @plsc.parallel_loop(0, K // L, unroll=8)
            def _(i):
                off = i * L
                pv = ib[pl.ds(off, L)]
                a = pv & mask16
                bb = (pv >> 16) & mask16
                va = plsc.load_gather(rp, [zrow, a])
                vb = plsc.load_gather(rp, [zrow, bb])
                o2[0, pl.ds(off, L)] = (va + vb) * 0.5
                wa = plsc.load_gather(rp, [orow, a])
                wb = plsc.load_gather(rp, [orow, bb])
                o2[1, pl.ds(off, L)] = (wa + wb) * 0.5

            odescs[c] = pltpu.async_copy(
                o2, out_ref.at[b0, pl.ds(c0, 2), pl.ds(VOLD + c * K, K)],
                sol)
        odescs[NCHUNK - 2].wait()
        odescs[NCHUNK - 1].wait()
        od.wait()
        return carry

    lax.fori_loop(0, ROW_PAIRS_PER_W, group, 0)


_main_call = pl.kernel(
    _main_body,
    out_type=jax.ShapeDtypeStruct((4, 64, NV), jnp.float32),
    mesh=_MESH,
    compiler_params=_CPARAMS,
    scratch_types=[
        pltpu.VMEM((2, ROWBUF), jnp.float32),
        pltpu.VMEM((K,), jnp.int32),
        pltpu.VMEM((K,), jnp.int32),
        pltpu.VMEM((2, K), jnp.float32),
        pltpu.VMEM((2, K), jnp.float32),
        pltpu.VMEM((512,), jnp.float32),
        pltpu.SemaphoreType.DMA,
        pltpu.SemaphoreType.DMA,
        pltpu.SemaphoreType.DMA,
        pltpu.SemaphoreType.DMA,
    ],
)


def kernel(x, F_prev, idx0, idx1, idx2):
    B, C, nv_prev = x.shape
    ext = _pairs_call(F_prev.reshape(-1), idx0, idx1, idx2)
    xt = x[:, :, VOLD:].reshape(-1)
    out = _main_call(x, ext, xt)

    ev = lax.dynamic_slice_in_dim(ext, NV_PAD, 2)
    a = (ev & 0xFFFF) - BIAS
    bb = (jnp.right_shift(ev, 16) & 0xFFFF) - BIAS
    cols = []
    for j in range(2):
        xa = lax.dynamic_slice_in_dim(x, jnp.maximum(a[j], 0), 1, axis=2)
        xb = lax.dynamic_slice_in_dim(x, jnp.maximum(bb[j], 0), 1, axis=2)
        col = (xa + xb) * 0.5
        col = jnp.where(ev[j] == 0, jnp.float32(1.0), col)
        cols.append(col)
    tail = jnp.concatenate(cols, axis=2)
    return lax.dynamic_update_slice(out, tail, (0, 0, NV - 2))


# speedup vs baseline: 1.2009x; 1.2009x over previous
import jax
import jax.numpy as jnp
import numpy as np
from jax import lax
from jax.experimental import pallas as pl
from jax.experimental.pallas import tpu as pltpu
from jax.experimental.pallas import tpu_sc as plsc

NV_PREV = 40962
NV = 163842
NV_PAD = NV - NV_PREV
NF_PREV = 81920

NC, NS, L = 2, 16, 16
NW = NC * NS

BIAS = 128
ROWBUF = 41216
EXT_LEN = 126976
FILL_PER_TILE = EXT_LEN // NS
OUT_PER_TILE = EXT_LEN // NW
FACES_PER_TILE = NF_PREV // NS
SCAT_ROWS = FACES_PER_TILE // 128

K = 6144
NCHUNK = NV_PAD // K
ROW_PAIRS_PER_W = (256 // 2) // NW
VOLD = 40960

_SELF0 = int(np.int32(np.uint32(((40960 + BIAS) << 16) | (40960 + BIAS))))
_SELF1 = int(np.int32(np.uint32(((40961 + BIAS) << 16) | (40961 + BIAS))))

_MESH = plsc.VectorSubcoreMesh(
    core_axis_name="c", subcore_axis_name="s", num_cores=NC, num_subcores=NS)

_CPARAMS = pltpu.CompilerParams(
    needs_layout_passes=False, use_tc_tiling_on_sc=True)


def _pairs_body(f0_ref, f1_ref, f2_ref, i0_ref, i1_ref, i2_ref, out_ref,
                fb0, fb1, fb2, ibuf, vb01, vb12, dbuf, zbuf, shared, sem):
    cid = lax.axis_index("c")
    sid = lax.axis_index("s")
    iota = lax.iota(jnp.int32, L)
    zeros = jnp.zeros((L,), jnp.int32)
    fbase = sid * FACES_PER_TILE

    pltpu.sync_copy(f0_ref.at[pl.ds(fbase, FACES_PER_TILE)], fb0)
    pltpu.sync_copy(f1_ref.at[pl.ds(fbase, FACES_PER_TILE)], fb1)
    pltpu.sync_copy(f2_ref.at[pl.ds(fbase, FACES_PER_TILE)], fb2)

    def vloop(i, carry):
        sl = pl.ds(i * L, L)
        f0 = fb0[sl]
        f1 = fb1[sl]
        f2 = fb2[sl]
        v01 = (f0 + BIAS) | ((f1 + BIAS) << 16)
        v12 = (f1 + BIAS) | ((f2 + BIAS) << 16)
        r = i // 8
        c = (i % 8) * L
        vb01[r, pl.ds(c, L)] = v01
        vb12[r, pl.ds(c, L)] = v12
        return carry

    lax.fori_loop(0, FACES_PER_TILE // L, vloop, 0)

    def zloop(i, carry):
        zbuf[pl.ds(i * L, L)] = zeros
        return carry

    lax.fori_loop(0, FILL_PER_TILE // L, zloop, 0)

    @pl.when(sid == 0)
    def _():
        patch = jnp.where(iota == 0, jnp.int32(_SELF0),
                          jnp.where(iota == 1, jnp.int32(_SELF1), 0))
        zbuf[pl.ds(0, L)] = patch

    pltpu.sync_copy(zbuf, shared.at[pl.ds(sid * FILL_PER_TILE, FILL_PER_TILE)])
    plsc.subcore_barrier()

    for idx_ref, vb in ((i0_ref, vb01), (i1_ref, vb12), (i2_ref, vb12)):
        pltpu.sync_copy(idx_ref.at[pl.ds(fbase, FACES_PER_TILE)], ibuf)

        def dloop(i, carry):
            dv = ibuf[pl.ds(i * L, L)] - (NV_PREV - 2)
            r = i // 8
            c = (i % 8) * L
            dbuf[r, pl.ds(c, L)] = dv
            return carry

        lax.fori_loop(0, FACES_PER_TILE // L, dloop, 0)
        descs = [pltpu.async_copy(vb.at[j], shared.at[dbuf.at[j]], sem)
                 for j in range(SCAT_ROWS)]
        for d in descs:
            d.wait()
        plsc.subcore_barrier()

    wid = sid * NC + cid
    obase = wid * OUT_PER_TILE
    pltpu.sync_copy(shared.at[pl.ds(obase, OUT_PER_TILE)],
                    zbuf.at[pl.ds(0, OUT_PER_TILE)])
    pltpu.sync_copy(zbuf.at[pl.ds(0, OUT_PER_TILE)],
                    out_ref.at[pl.ds(obase, OUT_PER_TILE)])


_pairs_call = pl.kernel(
    _pairs_body,
    out_type=jax.ShapeDtypeStruct((EXT_LEN,), jnp.int32),
    mesh=_MESH,
    compiler_params=_CPARAMS,
    scratch_types=[
        pltpu.VMEM((FACES_PER_TILE,), jnp.int32),
        pltpu.VMEM((FACES_PER_TILE,), jnp.int32),
        pltpu.VMEM((FACES_PER_TILE,), jnp.int32),
        pltpu.VMEM((FACES_PER_TILE,), jnp.int32),
        pltpu.VMEM((SCAT_ROWS, 128), jnp.int32),
        pltpu.VMEM((SCAT_ROWS, 128), jnp.int32),
        pltpu.VMEM((SCAT_ROWS, 128), jnp.int32),
        pltpu.VMEM((FILL_PER_TILE,), jnp.int32),
        pltpu.VMEM_SHARED((EXT_LEN,), jnp.int32),
        pltpu.SemaphoreType.DMA,
    ],
)


def _main_body(x_ref, ext_ref, xt_ref, out_ref,
               rp, ib0, ib1, oa, ob, xtb,
               si0, si1, sol, sold):
    cid = lax.axis_index("c")
    sid = lax.axis_index("s")
    iota = lax.iota(jnp.int32, L)
    wid = sid * NC + cid
    ones = jnp.full((L,), 1.0, jnp.float32)
    mask16 = jnp.int32(0xFFFF)
    zrow = jnp.zeros((L,), jnp.int32)
    orow = jnp.full((L,), 1, jnp.int32)
    ibufs = (ib0, ib1)
    obufs = (oa, ob)
    isems = (si0, si1)

    pltpu.sync_copy(xt_ref, xtb)

    def group(k, carry):
        r0 = 2 * (wid * ROW_PAIRS_PER_W + k)
        b0, c0 = r0 // 64, r0 % 64
        rp[0, pl.ds(0, L)] = ones
        rp[1, pl.ds(0, L)] = ones
        pltpu.sync_copy(x_ref.at[b0, pl.ds(c0, 2), pl.ds(0, VOLD)],
                        rp.at[:, pl.ds(BIAS, VOLD)])
        pos = iota + (BIAS + VOLD)
        msk2 = iota < 2
        v0 = plsc.load_gather(xtb, [jnp.minimum(2 * r0 + iota, 511)])
        v1 = plsc.load_gather(xtb, [jnp.minimum(2 * r0 + 2 + iota, 511)])
        plsc.store_scatter(rp, [zrow, pos], v0, mask=msk2)
        plsc.store_scatter(rp, [orow, pos], v1, mask=msk2)
        od = pltpu.async_copy(rp.at[:, pl.ds(BIAS, VOLD)],
                              out_ref.at[b0, pl.ds(c0, 2), pl.ds(0, VOLD)],
                              sold)

        idescs = {}
        odescs = {}
        idescs[0] = pltpu.async_copy(ext_ref.at[pl.ds(0, K)], ibufs[0], isems[0])
        for c in range(NCHUNK):
            cur = c % 2
            if c + 1 < NCHUNK:
                idescs[c + 1] = pltpu.async_copy(
                    ext_ref.at[pl.ds((c + 1) * K, K)],
                    ibufs[(c + 1) % 2], isems[(c + 1) % 2])
            idescs[c].wait()
            if c >= 2:
                odescs[c - 2].wait()
            ib = ibufs[cur]
            o2 = obufs[cur]

            @plsc.parallel_loop(0, K // L, unroll=8)
            def _(i):
                off = i * L
                pv = ib[pl.ds(off, L)]
                a = pv & mask16
                bb = (pv >> 16) & mask16
                va = plsc.load_gather(rp, [zrow, a])
                vb = plsc.load_gather(rp, [zrow, bb])
                o2[0, pl.ds(off, L)] = (va + vb) * 0.5
                wa = plsc.load_gather(rp, [orow, a])
                wb = plsc.load_gather(rp, [orow, bb])
                o2[1, pl.ds(off, L)] = (wa + wb) * 0.5

            odescs[c] = pltpu.async_copy(
                o2, out_ref.at[b0, pl.ds(c0, 2), pl.ds(VOLD + c * K, K)],
                sol)
        odescs[NCHUNK - 2].wait()
        odescs[NCHUNK - 1].wait()
        od.wait()
        return carry

    lax.fori_loop(0, ROW_PAIRS_PER_W, group, 0)


_main_call = pl.kernel(
    _main_body,
    out_type=jax.ShapeDtypeStruct((4, 64, NV), jnp.float32),
    mesh=_MESH,
    compiler_params=_CPARAMS,
    scratch_types=[
        pltpu.VMEM((2, ROWBUF), jnp.float32),
        pltpu.VMEM((K,), jnp.int32),
        pltpu.VMEM((K,), jnp.int32),
        pltpu.VMEM((2, K), jnp.float32),
        pltpu.VMEM((2, K), jnp.float32),
        pltpu.VMEM((512,), jnp.float32),
        pltpu.SemaphoreType.DMA,
        pltpu.SemaphoreType.DMA,
        pltpu.SemaphoreType.DMA,
        pltpu.SemaphoreType.DMA,
    ],
)


def kernel(x, F_prev, idx0, idx1, idx2):
    B, C, nv_prev = x.shape
    ext = _pairs_call(F_prev[:, 0], F_prev[:, 1], F_prev[:, 2],
                      idx0, idx1, idx2)
    xt = x[:, :, VOLD:].reshape(-1)
    out = _main_call(x, ext, xt)

    ev = lax.dynamic_slice_in_dim(ext, NV_PAD, 2)
    a = (ev & 0xFFFF) - BIAS
    bb = (jnp.right_shift(ev, 16) & 0xFFFF) - BIAS
    cols = []
    for j in range(2):
        xa = lax.dynamic_slice_in_dim(x, jnp.maximum(a[j], 0), 1, axis=2)
        xb = lax.dynamic_slice_in_dim(x, jnp.maximum(bb[j], 0), 1, axis=2)
        col = (xa + xb) * 0.5
        col = jnp.where(ev[j] == 0, jnp.float32(1.0), col)
        cols.append(col)
    tail = jnp.concatenate(cols, axis=2)
    return lax.dynamic_update_slice(out, tail, (0, 0, NV - 2))
